# baseline (speedup 1.0000x reference)
import functools

import jax
import jax.numpy as jnp
from jax import lax
from jax.experimental import pallas as pl
from jax.experimental.pallas import tpu as pltpu

try:
    from jax.experimental.pallas import tpu_sc as plsc
    _HAS_SC = True
except ImportError:
    plsc = None
    _HAS_SC = False

_D = 512
_B = 1024
_M = 100000
_K = 8
_G = 256
_MT = 2048
_MP = 100352
_NG = _MP // _G
_NT = _MP // _MT
_GPT = _MT // _G
_NEG = -1e30
_BIGI = 2 ** 30
_SCALE = _D ** (-0.5)
_TEMP = 5.0
_EPS = 1e-5


def _simskernel(q_ref, k_ref, sims_ref, gmax_ref, qh_ref, ql_ref):
    i = pl.program_id(0)

    @pl.when(i == 0)
    def _():
        q = q_ref[...]
        qnorm = jnp.sqrt(jnp.sum(q * q, axis=1, keepdims=True))
        qn = q / (qnorm + 1e-8)
        qh = qn.astype(jnp.bfloat16)
        qh_ref[...] = qh
        ql_ref[...] = (qn - qh.astype(jnp.float32)).astype(jnp.bfloat16)

    kb = k_ref[...]
    knorm = jnp.sqrt(jnp.sum(kb * kb, axis=1, keepdims=True))
    kbn = kb / (knorm + 1e-8)
    kh = kbn.astype(jnp.bfloat16)
    kl = (kbn - kh.astype(jnp.float32)).astype(jnp.bfloat16)
    dn = (((1,), (1,)), ((), ()))
    qh = qh_ref[...]
    s = lax.dot_general(qh, kh, dn, preferred_element_type=jnp.float32)
    s = s + lax.dot_general(qh, kl, dn, preferred_element_type=jnp.float32)
    s = s + lax.dot_general(ql_ref[...], kh, dn,
                            preferred_element_type=jnp.float32)
    col = i * _MT + lax.broadcasted_iota(jnp.int32, (1, _MT), 1)
    s = jnp.where(col < _M, s, _NEG)
    sims_ref[0] = s
    gvals = [jnp.max(s[:, g * _G:(g + 1) * _G], axis=1, keepdims=True)
             for g in range(_GPT)]
    gmax_ref[0] = jnp.concatenate(gvals, axis=1)


def _phase_a(query, keys_p, interpret=False):
    return pl.pallas_call(
        _simskernel,
        grid=(_NT,),
        in_specs=[
            pl.BlockSpec((_B, _D), lambda i: (0, 0)),
            pl.BlockSpec((_MT, _D), lambda i: (i, 0)),
        ],
        out_specs=[
            pl.BlockSpec((1, _B, _MT), lambda i: (i, 0, 0)),
            pl.BlockSpec((1, _B, _GPT), lambda i: (i, 0, 0)),
        ],
        out_shape=[
            jax.ShapeDtypeStruct((_NT, _B, _MT), jnp.float32),
            jax.ShapeDtypeStruct((_NT, _B, _GPT), jnp.float32),
        ],
        scratch_shapes=[pltpu.VMEM((_B, _D), jnp.bfloat16),
                        pltpu.VMEM((_B, _D), jnp.bfloat16)],
        interpret=interpret,
    )(query, keys_p)


def _topgroups_kernel(gmax_ref, flat_ref, gid_ref):
    v = gmax_ref[...]
    colv = lax.broadcasted_iota(jnp.int32, (_B, _NG), 1)
    rowb = lax.broadcasted_iota(jnp.int32, (_B, 1), 0)
    gids = []
    for _ in range(_K):
        m = jnp.max(v, axis=1, keepdims=True)
        sel = jnp.min(jnp.where(v >= m, colv, _BIGI), axis=1, keepdims=True)
        gids.append(sel)
        v = jnp.where(colv == sel, _NEG, v)
    gid = jnp.concatenate(gids, axis=1)
    gid_ref[...] = gid
    flat_ref[...] = (gid >> 3) * (_B * _GPT) + rowb * _GPT + (gid & 7)


def _phase_b(gmax, interpret=False):
    return pl.pallas_call(
        _topgroups_kernel,
        out_shape=[
            jax.ShapeDtypeStruct((_B, _K), jnp.int32),
            jax.ShapeDtypeStruct((_B, _K), jnp.int32),
        ],
        interpret=interpret,
    )(gmax)


def _topk_kernel(cand_ref, gid_ref, vals_ref, idx_ref):
    v = cand_ref[...]
    ji = lax.broadcasted_iota(jnp.int32, (_B, _G), 1)
    cols = []
    for k in range(_K):
        cols.append(gid_ref[:, k:k + 1] * _G + ji)
    gidx = jnp.concatenate(cols, axis=1)
    vals, idxs = [], []
    for _ in range(_K):
        m = jnp.max(v, axis=1, keepdims=True)
        sel = jnp.min(jnp.where(v >= m, gidx, _BIGI), axis=1, keepdims=True)
        vals.append(m)
        idxs.append(sel)
        v = jnp.where(gidx == sel, _NEG, v)
    vals_ref[...] = jnp.concatenate(vals, axis=1)
    idx_ref[...] = jnp.minimum(jnp.concatenate(idxs, axis=1), _M - 1)


def _phase_d(cand, gid, interpret=False):
    return pl.pallas_call(
        _topk_kernel,
        out_shape=[
            jax.ShapeDtypeStruct((_B, _K), jnp.float32),
            jax.ShapeDtypeStruct((_B, _K), jnp.int32),
        ],
        interpret=interpret,
    )(cand, gid)


def _sc_gather(table, idx, rows_per_buf):
    n, d = idx.shape[0], table.shape[1]
    info = plsc.get_sparse_core_info()
    nw = info.num_cores * info.num_subcores
    b_per_w = n // nw
    nchunks = b_per_w // rows_per_buf
    mesh = plsc.VectorSubcoreMesh(core_axis_name="c", subcore_axis_name="s")

    @functools.partial(
        pl.kernel, mesh=mesh,
        out_type=jax.ShapeDtypeStruct((n, d), jnp.float32),
        scratch_types=[
            pltpu.VMEM((rows_per_buf,), jnp.int32),
            pltpu.VMEM((rows_per_buf, d), jnp.float32),
            pltpu.SemaphoreType.DMA,
        ],
    )
    def k(table_hbm, idx_hbm, out_hbm, idx_v, rows_v, sem):
        wid = lax.axis_index("s") * info.num_cores + lax.axis_index("c")
        base = wid * b_per_w
        for c in range(nchunks):
            off = base + c * rows_per_buf
            pltpu.sync_copy(idx_hbm.at[pl.ds(off, rows_per_buf)], idx_v)
            pltpu.async_copy(table_hbm.at[idx_v], rows_v, sem).wait()
            pltpu.sync_copy(rows_v, out_hbm.at[pl.ds(off, rows_per_buf)])

    return k(table, idx)


def _fusion_kernel(q_ref, r_ref, tv_ref, wq_ref, bq_ref, wk_ref, bk_ref,
                   wv_ref, bv_ref, wo_ref, bo_ref, wg1_ref, wg2_ref, bg_ref,
                   g_ref, be_ref, out_ref):
    q = q_ref[...]
    tv = tv_ref[...]
    mask = tv > 0.0

    Q = jnp.dot(q, wq_ref[...], preferred_element_type=jnp.float32) + bq_ref[...]
    A = lax.dot_general(Q, wk_ref[...], (((1,), (1,)), ((), ())),
                        preferred_element_type=jnp.float32)
    qbk = jnp.sum(Q * bk_ref[...], axis=1, keepdims=True)

    scores = []
    for k in range(_K):
        rk = r_ref[:, k, :]
        scores.append(jnp.sum(A * rk, axis=1, keepdims=True))
    s = (jnp.concatenate(scores, axis=1) + qbk) * _SCALE

    valid = jnp.max(jnp.where(mask, 1.0, 0.0), axis=1, keepdims=True) > 0.0
    sm = jnp.where(mask, s, _NEG)
    smax = jnp.max(sm, axis=1, keepdims=True)
    e = jnp.where(mask, jnp.exp(sm - smax), 0.0)
    denom = jnp.sum(e, axis=1, keepdims=True)
    w = jnp.where(valid, e / jnp.where(valid, denom, 1.0), 0.0)
    sw = jnp.sum(w, axis=1, keepdims=True)

    rbar = jnp.zeros_like(q)
    for k in range(_K):
        rbar = rbar + w[:, k:k + 1] * r_ref[:, k, :]
    mem = jnp.dot(rbar, wv_ref[...], preferred_element_type=jnp.float32) \
        + sw * bv_ref[...]
    mem = jnp.dot(mem, wo_ref[...], preferred_element_type=jnp.float32) \
        + bo_ref[...]

    max_sim = jnp.where(valid, tv[:, 0:1], 0.0)
    glin = jnp.sum(q * wg1_ref[...], axis=1, keepdims=True) \
        + jnp.sum(mem * wg2_ref[...], axis=1, keepdims=True) + bg_ref[...]
    gate = 1.0 / (1.0 + jnp.exp(-glin))
    conf = 1.0 / (1.0 + jnp.exp(-_TEMP * max_sim))
    gate = gate * conf

    out = q + gate * mem
    out = jnp.where(valid, out, q)

    mu = jnp.mean(out, axis=1, keepdims=True)
    d0 = out - mu
    var = jnp.mean(d0 * d0, axis=1, keepdims=True)
    out_ref[...] = d0 * lax.rsqrt(var + _EPS) * g_ref[...] + be_ref[...]


def _phase_f(query, retrieved, top_vals, Wq, bq, Wk, bk, Wv, bv, Wo, bo,
             Wg, bg, ln_gamma, ln_beta, interpret=False):
    row = lambda x: x.reshape(1, -1)
    return pl.pallas_call(
        _fusion_kernel,
        out_shape=jax.ShapeDtypeStruct((_B, _D), jnp.float32),
        interpret=interpret,
    )(query, retrieved, top_vals, Wq, row(bq), Wk, row(bk), Wv, row(bv),
      Wo, row(bo), row(Wg[:_D, 0]), row(Wg[_D:, 0]), row(bg),
      row(ln_gamma), row(ln_beta))


def kernel(query, memory_keys, memory_values, Wq, bq, Wk, bk, Wv, bv,
           Wo, bo, Wg, bg, ln_gamma, ln_beta):
    keys_p = jnp.pad(memory_keys, ((0, _MP - _M), (0, 0)))

    sims, gmax3 = _phase_a(query, keys_p)
    gmax = jnp.transpose(gmax3, (1, 0, 2)).reshape(_B, _NG)
    flat_ids, gid = _phase_b(gmax)

    cand = _sc_gather(sims.reshape(_NT * _B * _GPT, _G),
                      flat_ids.reshape(_B * _K), rows_per_buf=256)
    top_vals, top_idx = _phase_d(cand.reshape(_B, _K * _G), gid)

    retrieved = _sc_gather(memory_values, top_idx.reshape(_B * _K),
                           rows_per_buf=128)

    return _phase_f(query, retrieved.reshape(_B, _K, _D), top_vals,
                    Wq, bq, Wk, bk, Wv, bv, Wo, bo, Wg, bg,
                    ln_gamma, ln_beta)


# speedup vs baseline: 1.5689x; 1.5689x over previous
import functools

import jax
import jax.numpy as jnp
from jax import lax
from jax.experimental import pallas as pl
from jax.experimental.pallas import tpu as pltpu
from jax.experimental.pallas import tpu_sc as plsc

_D = 512
_B = 1024
_M = 100000
_K = 8
_GF = 8
_G = 256
_MT = 2048
_MP = 100352
_NG = _MP // _G
_NF = _MP // _GF
_NT = _MP // _MT
_GPT = _MT // _G
_FPT = _MT // _GF
_NEG = -1e30
_BIGI = 2 ** 30
_SCALE = _D ** (-0.5)
_TEMP = 5.0
_EPS = 1e-5


def _simskernel(q_ref, k_ref, g8_ref, g256_ref, qn_ref):
    i = pl.program_id(0)

    @pl.when(i == 0)
    def _():
        q = q_ref[...]
        qnorm = jnp.sqrt(jnp.sum(q * q, axis=1, keepdims=True))
        qn_ref[...] = q / (qnorm + 1e-8)

    kb = k_ref[...]
    knorm = jnp.sqrt(jnp.sum(kb * kb, axis=1, keepdims=True))
    kbn = kb / (knorm + 1e-8)
    s = lax.dot_general(qn_ref[...], kbn, (((1,), (1,)), ((), ())),
                        preferred_element_type=jnp.float32)
    col = i * _MT + lax.broadcasted_iota(jnp.int32, (1, _MT), 1)
    s = jnp.where(col < _M, s, _NEG)
    m8 = s[:, 0:_FPT]
    for r in range(1, _GF):
        m8 = jnp.maximum(m8, s[:, r * _FPT:(r + 1) * _FPT])
    g8_ref[0] = m8
    t = m8
    w = _FPT
    while w > _GPT:
        w //= 2
        t = jnp.maximum(t[:, :w], t[:, w:2 * w])
    g256_ref[0] = t


def _phase_a(query, keys_p, interpret=False):
    return pl.pallas_call(
        _simskernel,
        grid=(_NT,),
        in_specs=[
            pl.BlockSpec((_B, _D), lambda i: (0, 0)),
            pl.BlockSpec((_MT, _D), lambda i: (i, 0)),
        ],
        out_specs=[
            pl.BlockSpec((1, _B, _FPT), lambda i: (i, 0, 0)),
            pl.BlockSpec((1, _B, _GPT), lambda i: (i, 0, 0)),
            pl.BlockSpec((_B, _D), lambda i: (0, 0)),
        ],
        out_shape=[
            jax.ShapeDtypeStruct((_NT, _B, _FPT), jnp.float32),
            jax.ShapeDtypeStruct((_NT, _B, _GPT), jnp.float32),
            jax.ShapeDtypeStruct((_B, _D), jnp.float32),
        ],
        interpret=interpret,
    )(query, keys_p)


_BR = 128


def _topgroups_kernel(gmax_ref, flat_ref, gid_ref):
    v = gmax_ref[...]
    colv = lax.broadcasted_iota(jnp.int32, (_BR, _NG), 1)
    rowb = pl.program_id(0) * _BR + \
        lax.broadcasted_iota(jnp.int32, (_BR, 1), 0)
    gids = []
    for _ in range(_K):
        m = jnp.max(v, axis=1, keepdims=True)
        sel = jnp.min(jnp.where(v >= m, colv, _BIGI), axis=1, keepdims=True)
        gids.append(sel)
        v = jnp.where(colv == sel, _NEG, v)
    gid = jnp.concatenate(gids, axis=1)
    gid_ref[...] = gid
    flat_ref[...] = (gid >> 3) * _B + rowb


def _phase_b(gmax, interpret=False):
    return pl.pallas_call(
        _topgroups_kernel,
        grid=(_B // _BR,),
        in_specs=[pl.BlockSpec((_BR, _NG), lambda i: (i, 0))],
        out_specs=[pl.BlockSpec((_BR, _K), lambda i: (i, 0)),
                   pl.BlockSpec((_BR, _K), lambda i: (i, 0))],
        out_shape=[
            jax.ShapeDtypeStruct((_B, _K), jnp.int32),
            jax.ShapeDtypeStruct((_B, _K), jnp.int32),
        ],
        interpret=interpret,
    )(gmax)


def _finesel_kernel(cand_ref, gid_ref, fid_ref, rows_ref):
    v = cand_ref[...]
    ji = lax.broadcasted_iota(jnp.int32, (_BR, _FPT), 1)
    cols = []
    for k in range(_K):
        cols.append((gid_ref[:, k:k + 1] >> 3) * _FPT + ji)
    fidx = jnp.concatenate(cols, axis=1)
    fids = []
    for _ in range(_K):
        m = jnp.max(v, axis=1, keepdims=True)
        sel = jnp.min(jnp.where(v >= m, fidx, _BIGI), axis=1, keepdims=True)
        fids.append(sel)
        v = jnp.where(fidx == sel, _NEG, v)
    fid_ref[...] = jnp.concatenate(fids, axis=1)
    rcols = []
    for k in range(_K):
        base = (fids[k] >> 8) * _MT + (fids[k] & (_FPT - 1))
        for r in range(_GF):
            rcols.append(base + r * _FPT)
    rows_ref[...] = jnp.minimum(jnp.concatenate(rcols, axis=1), _M - 1)


def _phase_d1(cand, gid, interpret=False):
    return pl.pallas_call(
        _finesel_kernel,
        grid=(_B // _BR,),
        in_specs=[pl.BlockSpec((_BR, _K * _FPT), lambda i: (i, 0)),
                  pl.BlockSpec((_BR, _K), lambda i: (i, 0))],
        out_specs=[pl.BlockSpec((_BR, _K), lambda i: (i, 0)),
                   pl.BlockSpec((_BR, 64), lambda i: (i, 0))],
        out_shape=[jax.ShapeDtypeStruct((_B, _K), jnp.int32),
                   jax.ShapeDtypeStruct((_B, 64), jnp.int32)],
        interpret=interpret,
    )(cand, gid)


def _topk_kernel(dots_ref, nsq_ref, fid_ref, vals_ref, idx_ref):
    v = dots_ref[...] / (jnp.sqrt(nsq_ref[...]) + 1e-8)
    cols = []
    for k in range(_K):
        fk = fid_ref[:, k:k + 1]
        base = (fk >> 8) * _MT + (fk & (_FPT - 1))
        for r in range(_GF):
            cols.append(base + r * _FPT)
    gidx = jnp.concatenate(cols, axis=1)
    vals, idxs = [], []
    for _ in range(_K):
        m = jnp.max(v, axis=1, keepdims=True)
        sel = jnp.min(jnp.where(v >= m, gidx, _BIGI), axis=1, keepdims=True)
        vals.append(m)
        idxs.append(sel)
        v = jnp.where(gidx == sel, _NEG, v)
    vals_ref[...] = jnp.concatenate(vals, axis=1)
    idx_ref[...] = jnp.minimum(jnp.concatenate(idxs, axis=1), _M - 1)


def _phase_d2(dots, nsq, fid, interpret=False):
    return pl.pallas_call(
        _topk_kernel,
        grid=(_B // _BR,),
        in_specs=[pl.BlockSpec((_BR, 64), lambda i: (i, 0)),
                  pl.BlockSpec((_BR, 64), lambda i: (i, 0)),
                  pl.BlockSpec((_BR, _K), lambda i: (i, 0))],
        out_specs=[pl.BlockSpec((_BR, _K), lambda i: (i, 0)),
                   pl.BlockSpec((_BR, _K), lambda i: (i, 0))],
        out_shape=[
            jax.ShapeDtypeStruct((_B, _K), jnp.float32),
            jax.ShapeDtypeStruct((_B, _K), jnp.int32),
        ],
        interpret=interpret,
    )(dots, nsq, fid)


def _sc_gather(table, idx, rows_per_buf):
    n, d = idx.shape[0], table.shape[1]
    info = plsc.get_sparse_core_info()
    nw = info.num_cores * info.num_subcores
    b_per_w = n // nw
    nchunks = b_per_w // rows_per_buf
    mesh = plsc.VectorSubcoreMesh(core_axis_name="c", subcore_axis_name="s")

    @functools.partial(
        pl.kernel, mesh=mesh,
        out_type=jax.ShapeDtypeStruct((n, d), jnp.float32),
        scratch_types=[
            pltpu.VMEM((rows_per_buf,), jnp.int32),
            pltpu.VMEM((rows_per_buf, d), jnp.float32),
            pltpu.SemaphoreType.DMA,
        ],
    )
    def k(table_hbm, idx_hbm, out_hbm, idx_v, rows_v, sem):
        wid = lax.axis_index("s") * info.num_cores + lax.axis_index("c")
        base = wid * b_per_w
        for c in range(nchunks):
            off = base + c * rows_per_buf
            pltpu.sync_copy(idx_hbm.at[pl.ds(off, rows_per_buf)], idx_v)
            pltpu.async_copy(table_hbm.at[idx_v], rows_v, sem).wait()
            pltpu.sync_copy(rows_v, out_hbm.at[pl.ds(off, rows_per_buf)])

    return k(table, idx)


def _sc_rowdot(ktab, rows_flat, qn_flat):
    info = plsc.get_sparse_core_info()
    nw = info.num_cores * info.num_subcores
    qpw = _B // nw
    rpw = qpw * 64
    nbody = rpw // 16
    mesh = plsc.VectorSubcoreMesh(core_axis_name="c", subcore_axis_name="s")

    @functools.partial(
        pl.kernel, mesh=mesh,
        out_type=[jax.ShapeDtypeStruct((_B * 64,), jnp.float32),
                  jax.ShapeDtypeStruct((_B * 64,), jnp.float32)],
        scratch_types=[
            pltpu.VMEM((rpw,), jnp.int32),
            pltpu.VMEM((qpw * _D,), jnp.float32),
            pltpu.VMEM((16, _D), jnp.float32),
            pltpu.VMEM((16,), jnp.float32),
            pltpu.VMEM((16,), jnp.float32),
            pltpu.SemaphoreType.DMA,
        ],
    )
    def k(ktab_hbm, rows_hbm, qn_hbm, dots_hbm, nsq_hbm,
          idx_v, qall, rowbuf, dbuf, nbuf, sem1):
        wid = lax.axis_index("s") * info.num_cores + lax.axis_index("c")
        base_q = wid * qpw
        pltpu.sync_copy(rows_hbm.at[pl.ds(wid * rpw, rpw)], idx_v)
        pltpu.sync_copy(qn_hbm.at[pl.ds(base_q * _D, qpw * _D)], qall)
        lane = lax.iota(jnp.int32, 16)

        def lane_sum(x):
            for sh in (8, 4, 2, 1):
                perm = (lane + sh) & 15
                x = x + x.at[perm].get(mode='promise_in_bounds')
            return x

        def body(t, carry):
            pltpu.async_copy(ktab_hbm.at[idx_v.at[pl.ds(t * 16, 16)]],
                             rowbuf, sem1).wait()
            ql = t >> 2
            dv = jnp.zeros((16,), jnp.float32)
            nv = jnp.zeros((16,), jnp.float32)
            for half in range(2):
                accs = [jnp.zeros((16,), jnp.float32) for _ in range(8)]
                accn = [jnp.zeros((16,), jnp.float32) for _ in range(8)]
                for c in range(_D // 16):
                    qv = qall[pl.ds(ql * _D + c * 16, 16)]
                    for r in range(8):
                        kv = rowbuf[half * 8 + r, pl.ds(c * 16, 16)]
                        accs[r] = accs[r] + kv * qv
                        accn[r] = accn[r] + kv * kv
                for r in range(8):
                    pos = half * 8 + r
                    dv = jnp.where(lane == pos, lane_sum(accs[r]), dv)
                    nv = jnp.where(lane == pos, lane_sum(accn[r]), nv)
            dbuf[...] = dv
            nbuf[...] = nv
            p0 = (wid * nbody + t) * 16
            pltpu.sync_copy(dbuf, dots_hbm.at[pl.ds(p0, 16)])
            pltpu.sync_copy(nbuf, nsq_hbm.at[pl.ds(p0, 16)])
            return carry

        lax.fori_loop(0, nbody, body, 0)

    return k(ktab, rows_flat, qn_flat)


def _fusion_kernel(q_ref, r_ref, tv_ref, wq_ref, bq_ref, wk_ref, bk_ref,
                   wv_ref, bv_ref, wo_ref, bo_ref, wg1_ref, wg2_ref, bg_ref,
                   g_ref, be_ref, out_ref):
    q = q_ref[...]
    tv = tv_ref[...]
    mask = tv > 0.0

    Q = jnp.dot(q, wq_ref[...], preferred_element_type=jnp.float32) + bq_ref[...]
    A = lax.dot_general(Q, wk_ref[...], (((1,), (1,)), ((), ())),
                        preferred_element_type=jnp.float32)
    qbk = jnp.sum(Q * bk_ref[...], axis=1, keepdims=True)

    scores = []
    for k in range(_K):
        rk = r_ref[:, k, :]
        scores.append(jnp.sum(A * rk, axis=1, keepdims=True))
    s = (jnp.concatenate(scores, axis=1) + qbk) * _SCALE

    valid = jnp.max(jnp.where(mask, 1.0, 0.0), axis=1, keepdims=True) > 0.0
    sm = jnp.where(mask, s, _NEG)
    smax = jnp.max(sm, axis=1, keepdims=True)
    e = jnp.where(mask, jnp.exp(sm - smax), 0.0)
    denom = jnp.sum(e, axis=1, keepdims=True)
    w = jnp.where(valid, e / jnp.where(valid, denom, 1.0), 0.0)
    sw = jnp.sum(w, axis=1, keepdims=True)

    rbar = jnp.zeros_like(q)
    for k in range(_K):
        rbar = rbar + w[:, k:k + 1] * r_ref[:, k, :]
    mem = jnp.dot(rbar, wv_ref[...], preferred_element_type=jnp.float32) \
        + sw * bv_ref[...]
    mem = jnp.dot(mem, wo_ref[...], preferred_element_type=jnp.float32) \
        + bo_ref[...]

    max_sim = jnp.where(valid, tv[:, 0:1], 0.0)
    glin = jnp.sum(q * wg1_ref[...], axis=1, keepdims=True) \
        + jnp.sum(mem * wg2_ref[...], axis=1, keepdims=True) + bg_ref[...]
    gate = 1.0 / (1.0 + jnp.exp(-glin))
    conf = 1.0 / (1.0 + jnp.exp(-_TEMP * max_sim))
    gate = gate * conf

    out = q + gate * mem
    out = jnp.where(valid, out, q)

    mu = jnp.mean(out, axis=1, keepdims=True)
    d0 = out - mu
    var = jnp.mean(d0 * d0, axis=1, keepdims=True)
    out_ref[...] = d0 * lax.rsqrt(var + _EPS) * g_ref[...] + be_ref[...]


def _phase_f(query, retrieved, top_vals, Wq, bq, Wk, bk, Wv, bv, Wo, bo,
             Wg, bg, ln_gamma, ln_beta, interpret=False):
    row = lambda x: x.reshape(1, -1)
    return pl.pallas_call(
        _fusion_kernel,
        out_shape=jax.ShapeDtypeStruct((_B, _D), jnp.float32),
        interpret=interpret,
    )(query, retrieved, top_vals, Wq, row(bq), Wk, row(bk), Wv, row(bv),
      Wo, row(bo), row(Wg[:_D, 0]), row(Wg[_D:, 0]), row(bg),
      row(ln_gamma), row(ln_beta))


def kernel(query, memory_keys, memory_values, Wq, bq, Wk, bk, Wv, bv,
           Wo, bo, Wg, bg, ln_gamma, ln_beta):
    keys_p = jnp.pad(memory_keys, ((0, _MP - _M), (0, 0)))

    gmax8, gmax256, qn = _phase_a(query, keys_p)
    gmax = jnp.transpose(gmax256, (1, 0, 2)).reshape(_B, _NG)
    flat_ids, gid = _phase_b(gmax)

    cand = _sc_gather(gmax8.reshape(_NT * _B, _FPT),
                      flat_ids.reshape(_B * _K), rows_per_buf=64)
    fid, rows64 = _phase_d1(cand.reshape(_B, _K * _FPT), gid)

    dots, nsq = _sc_rowdot(keys_p, rows64.reshape(_B * 64),
                           qn.reshape(_B * _D))
    top_vals, top_idx = _phase_d2(dots.reshape(_B, 64),
                                  nsq.reshape(_B, 64), fid)

    retrieved = _sc_gather(memory_values, top_idx.reshape(_B * _K),
                           rows_per_buf=128)

    return _phase_f(query, retrieved.reshape(_B, _K, _D), top_vals,
                    Wq, bq, Wk, bk, Wv, bv, Wo, bo, Wg, bg,
                    ln_gamma, ln_beta)


# speedup vs baseline: 1.6709x; 1.0650x over previous
import functools

import jax
import jax.numpy as jnp
from jax import lax
from jax.experimental import pallas as pl
from jax.experimental.pallas import tpu as pltpu
from jax.experimental.pallas import tpu_sc as plsc

_D = 512
_B = 1024
_M = 100000
_K = 8
_GF = 8
_G = 256
_MT = 2048
_MP = 100352
_NG = _MP // _G
_NF = _MP // _GF
_NT = _MP // _MT
_GPT = _MT // _G
_FPT = _MT // _GF
_NEG = -1e30
_BIGI = 2 ** 30
_SCALE = _D ** (-0.5)
_TEMP = 5.0
_EPS = 1e-5


def _simskernel(q_ref, k_ref, g8_ref, g256_ref, qn_ref):
    i = pl.program_id(0)

    @pl.when(i == 0)
    def _():
        q = q_ref[...]
        qnorm = jnp.sqrt(jnp.sum(q * q, axis=1, keepdims=True))
        qn_ref[...] = q / (qnorm + 1e-8)

    kb = k_ref[...]
    knorm = jnp.sqrt(jnp.sum(kb * kb, axis=1, keepdims=True))
    kbn = kb / (knorm + 1e-8)
    s = lax.dot_general(qn_ref[...], kbn, (((1,), (1,)), ((), ())),
                        preferred_element_type=jnp.float32)
    col = i * _MT + lax.broadcasted_iota(jnp.int32, (1, _MT), 1)
    s = jnp.where(col < _M, s, _NEG)
    m8 = s[:, 0:_FPT]
    for r in range(1, _GF):
        m8 = jnp.maximum(m8, s[:, r * _FPT:(r + 1) * _FPT])
    g8_ref[0] = m8
    t = m8
    w = _FPT
    while w > _GPT:
        w //= 2
        t = jnp.maximum(t[:, :w], t[:, w:2 * w])
    g256_ref[0] = t


def _phase_a(query, keys_p, interpret=False):
    return pl.pallas_call(
        _simskernel,
        grid=(_NT,),
        in_specs=[
            pl.BlockSpec((_B, _D), lambda i: (0, 0)),
            pl.BlockSpec((_MT, _D), lambda i: (i, 0)),
        ],
        out_specs=[
            pl.BlockSpec((1, _B, _FPT), lambda i: (i, 0, 0)),
            pl.BlockSpec((1, _B, _GPT), lambda i: (i, 0, 0)),
            pl.BlockSpec((_B, _D), lambda i: (0, 0)),
        ],
        out_shape=[
            jax.ShapeDtypeStruct((_NT, _B, _FPT), jnp.float32),
            jax.ShapeDtypeStruct((_NT, _B, _GPT), jnp.float32),
            jax.ShapeDtypeStruct((_B, _D), jnp.float32),
        ],
        interpret=interpret,
    )(query, keys_p)


_BR = 128


def _topgroups_kernel(gmax_ref, flat_ref, gid_ref):
    v = gmax_ref[...]
    colv = lax.broadcasted_iota(jnp.int32, (_BR, _NG), 1)
    rowb = pl.program_id(0) * _BR + \
        lax.broadcasted_iota(jnp.int32, (_BR, 1), 0)
    gids = []
    for _ in range(_K):
        m = jnp.max(v, axis=1, keepdims=True)
        sel = jnp.min(jnp.where(v >= m, colv, _BIGI), axis=1, keepdims=True)
        gids.append(sel)
        v = jnp.where(colv == sel, _NEG, v)
    gid = jnp.concatenate(gids, axis=1)
    gid_ref[...] = gid
    flat_ref[...] = (gid >> 3) * _B + rowb


def _phase_b(gmax, interpret=False):
    return pl.pallas_call(
        _topgroups_kernel,
        grid=(_B // _BR,),
        in_specs=[pl.BlockSpec((_BR, _NG), lambda i: (i, 0))],
        out_specs=[pl.BlockSpec((_BR, _K), lambda i: (i, 0)),
                   pl.BlockSpec((_BR, _K), lambda i: (i, 0))],
        out_shape=[
            jax.ShapeDtypeStruct((_B, _K), jnp.int32),
            jax.ShapeDtypeStruct((_B, _K), jnp.int32),
        ],
        interpret=interpret,
    )(gmax)


def _finesel_kernel(cand_ref, gid_ref, fid_ref, rows_ref):
    v = cand_ref[...]
    ji = lax.broadcasted_iota(jnp.int32, (_BR, _FPT), 1)
    cols = []
    for k in range(_K):
        cols.append((gid_ref[:, k:k + 1] >> 3) * _FPT + ji)
    fidx = jnp.concatenate(cols, axis=1)
    fids = []
    for _ in range(_K):
        m = jnp.max(v, axis=1, keepdims=True)
        sel = jnp.min(jnp.where(v >= m, fidx, _BIGI), axis=1, keepdims=True)
        fids.append(sel)
        v = jnp.where(fidx == sel, _NEG, v)
    fid_ref[...] = jnp.concatenate(fids, axis=1)
    rcols = []
    for k in range(_K):
        base = (fids[k] >> 8) * _MT + (fids[k] & (_FPT - 1))
        for r in range(_GF):
            rcols.append(base + r * _FPT)
    rows_ref[...] = jnp.minimum(jnp.concatenate(rcols, axis=1), _M - 1)


def _phase_d1(cand, gid, interpret=False):
    return pl.pallas_call(
        _finesel_kernel,
        grid=(_B // _BR,),
        in_specs=[pl.BlockSpec((_BR, _K * _FPT), lambda i: (i, 0)),
                  pl.BlockSpec((_BR, _K), lambda i: (i, 0))],
        out_specs=[pl.BlockSpec((_BR, _K), lambda i: (i, 0)),
                   pl.BlockSpec((_BR, 64), lambda i: (i, 0))],
        out_shape=[jax.ShapeDtypeStruct((_B, _K), jnp.int32),
                   jax.ShapeDtypeStruct((_B, 64), jnp.int32)],
        interpret=interpret,
    )(cand, gid)


def _topk_kernel(dots_ref, nsq_ref, fid_ref, vals_ref, idx_ref):
    v = dots_ref[...] / (jnp.sqrt(nsq_ref[...]) + 1e-8)
    cols = []
    for k in range(_K):
        fk = fid_ref[:, k:k + 1]
        base = (fk >> 8) * _MT + (fk & (_FPT - 1))
        for r in range(_GF):
            cols.append(base + r * _FPT)
    gidx = jnp.concatenate(cols, axis=1)
    vals, idxs = [], []
    for _ in range(_K):
        m = jnp.max(v, axis=1, keepdims=True)
        sel = jnp.min(jnp.where(v >= m, gidx, _BIGI), axis=1, keepdims=True)
        vals.append(m)
        idxs.append(sel)
        v = jnp.where(gidx == sel, _NEG, v)
    vals_ref[...] = jnp.concatenate(vals, axis=1)
    idx_ref[...] = jnp.minimum(jnp.concatenate(idxs, axis=1), _M - 1)


def _phase_d2(dots, nsq, fid, interpret=False):
    return pl.pallas_call(
        _topk_kernel,
        grid=(_B // _BR,),
        in_specs=[pl.BlockSpec((_BR, 64), lambda i: (i, 0)),
                  pl.BlockSpec((_BR, 64), lambda i: (i, 0)),
                  pl.BlockSpec((_BR, _K), lambda i: (i, 0))],
        out_specs=[pl.BlockSpec((_BR, _K), lambda i: (i, 0)),
                   pl.BlockSpec((_BR, _K), lambda i: (i, 0))],
        out_shape=[
            jax.ShapeDtypeStruct((_B, _K), jnp.float32),
            jax.ShapeDtypeStruct((_B, _K), jnp.int32),
        ],
        interpret=interpret,
    )(dots, nsq, fid)


def _sc_gather(table, idx, rows_per_buf):
    n, d = idx.shape[0], table.shape[1]
    info = plsc.get_sparse_core_info()
    nw = info.num_cores * info.num_subcores
    b_per_w = n // nw
    nchunks = b_per_w // rows_per_buf
    mesh = plsc.VectorSubcoreMesh(core_axis_name="c", subcore_axis_name="s")

    @functools.partial(
        pl.kernel, mesh=mesh,
        out_type=jax.ShapeDtypeStruct((n, d), jnp.float32),
        scratch_types=[
            pltpu.VMEM((rows_per_buf,), jnp.int32),
            pltpu.VMEM((rows_per_buf, d), jnp.float32),
            pltpu.SemaphoreType.DMA,
        ],
    )
    def k(table_hbm, idx_hbm, out_hbm, idx_v, rows_v, sem):
        wid = lax.axis_index("s") * info.num_cores + lax.axis_index("c")
        base = wid * b_per_w
        for c in range(nchunks):
            off = base + c * rows_per_buf
            pltpu.sync_copy(idx_hbm.at[pl.ds(off, rows_per_buf)], idx_v)
            pltpu.async_copy(table_hbm.at[idx_v], rows_v, sem).wait()
            pltpu.sync_copy(rows_v, out_hbm.at[pl.ds(off, rows_per_buf)])

    return k(table, idx)


def _sc_rowdot(ktab, rows_flat, qn_flat):
    info = plsc.get_sparse_core_info()
    nw = info.num_cores * info.num_subcores
    qpw = _B // nw
    rpw = qpw * 64
    nbody = rpw // 16
    mesh = plsc.VectorSubcoreMesh(core_axis_name="c", subcore_axis_name="s")

    @functools.partial(
        pl.kernel, mesh=mesh,
        out_type=[jax.ShapeDtypeStruct((_B * 64,), jnp.float32),
                  jax.ShapeDtypeStruct((_B * 64,), jnp.float32)],
        scratch_types=[
            pltpu.VMEM((rpw,), jnp.int32),
            pltpu.VMEM((qpw * _D,), jnp.float32),
            pltpu.VMEM((16, _D), jnp.float32),
            pltpu.VMEM((16, _D), jnp.float32),
            pltpu.VMEM((16,), jnp.float32),
            pltpu.VMEM((16,), jnp.float32),
            pltpu.SemaphoreType.DMA,
            pltpu.SemaphoreType.DMA,
        ],
    )
    def k(ktab_hbm, rows_hbm, qn_hbm, dots_hbm, nsq_hbm,
          idx_v, qall, rowbufa, rowbufb, dbuf, nbuf, sem1, sem2):
        wid = lax.axis_index("s") * info.num_cores + lax.axis_index("c")
        base_q = wid * qpw
        pltpu.sync_copy(rows_hbm.at[pl.ds(wid * rpw, rpw)], idx_v)
        pltpu.sync_copy(qn_hbm.at[pl.ds(base_q * _D, qpw * _D)], qall)
        lane = lax.iota(jnp.int32, 16)

        def lane_sum(x):
            for sh in (8, 4, 2, 1):
                perm = (lane + sh) & 15
                x = x + x.at[perm].get(mode='promise_in_bounds')
            return x

        def compute(t, rowbuf):
            ql = t >> 2
            dv = jnp.zeros((16,), jnp.float32)
            nv = jnp.zeros((16,), jnp.float32)
            for half in range(2):
                accs = [jnp.zeros((16,), jnp.float32) for _ in range(8)]
                accn = [jnp.zeros((16,), jnp.float32) for _ in range(8)]
                for c in range(_D // 16):
                    qv = qall[pl.ds(ql * _D + c * 16, 16)]
                    for r in range(8):
                        kv = rowbuf[half * 8 + r, pl.ds(c * 16, 16)]
                        accs[r] = accs[r] + kv * qv
                        accn[r] = accn[r] + kv * kv
                for r in range(8):
                    pos = half * 8 + r
                    dv = jnp.where(lane == pos, lane_sum(accs[r]), dv)
                    nv = jnp.where(lane == pos, lane_sum(accn[r]), nv)
            dbuf[...] = dv
            nbuf[...] = nv
            p0 = (wid * nbody + t) * 16
            pltpu.sync_copy(dbuf, dots_hbm.at[pl.ds(p0, 16)])
            pltpu.sync_copy(nbuf, nsq_hbm.at[pl.ds(p0, 16)])

        def gather(t, buf, sem):
            return pltpu.make_async_copy(
                ktab_hbm.at[idx_v.at[pl.ds(t * 16, 16)]], buf, sem)

        gather(0, rowbufa, sem1).start()

        def body(u, carry):
            t0 = u * 2
            gather(t0, rowbufa, sem1).wait()
            gather(t0 + 1, rowbufb, sem2).start()
            compute(t0, rowbufa)
            gather(t0 + 1, rowbufb, sem2).wait()

            @pl.when(t0 + 2 < nbody)
            def _():
                gather(t0 + 2, rowbufa, sem1).start()
            compute(t0 + 1, rowbufb)
            return carry

        lax.fori_loop(0, nbody // 2, body, 0)

    return k(ktab, rows_flat, qn_flat)


def _fusion_kernel(q_ref, r_ref, tv_ref, wq_ref, bq_ref, wk_ref, bk_ref,
                   wv_ref, bv_ref, wo_ref, bo_ref, wg1_ref, wg2_ref, bg_ref,
                   g_ref, be_ref, out_ref):
    q = q_ref[...]
    tv = tv_ref[...]
    mask = tv > 0.0

    Q = jnp.dot(q, wq_ref[...], preferred_element_type=jnp.float32) + bq_ref[...]
    A = lax.dot_general(Q, wk_ref[...], (((1,), (1,)), ((), ())),
                        preferred_element_type=jnp.float32)
    qbk = jnp.sum(Q * bk_ref[...], axis=1, keepdims=True)

    scores = []
    for k in range(_K):
        rk = r_ref[:, k, :]
        scores.append(jnp.sum(A * rk, axis=1, keepdims=True))
    s = (jnp.concatenate(scores, axis=1) + qbk) * _SCALE

    valid = jnp.max(jnp.where(mask, 1.0, 0.0), axis=1, keepdims=True) > 0.0
    sm = jnp.where(mask, s, _NEG)
    smax = jnp.max(sm, axis=1, keepdims=True)
    e = jnp.where(mask, jnp.exp(sm - smax), 0.0)
    denom = jnp.sum(e, axis=1, keepdims=True)
    w = jnp.where(valid, e / jnp.where(valid, denom, 1.0), 0.0)
    sw = jnp.sum(w, axis=1, keepdims=True)

    rbar = jnp.zeros_like(q)
    for k in range(_K):
        rbar = rbar + w[:, k:k + 1] * r_ref[:, k, :]
    mem = jnp.dot(rbar, wv_ref[...], preferred_element_type=jnp.float32) \
        + sw * bv_ref[...]
    mem = jnp.dot(mem, wo_ref[...], preferred_element_type=jnp.float32) \
        + bo_ref[...]

    max_sim = jnp.where(valid, tv[:, 0:1], 0.0)
    glin = jnp.sum(q * wg1_ref[...], axis=1, keepdims=True) \
        + jnp.sum(mem * wg2_ref[...], axis=1, keepdims=True) + bg_ref[...]
    gate = 1.0 / (1.0 + jnp.exp(-glin))
    conf = 1.0 / (1.0 + jnp.exp(-_TEMP * max_sim))
    gate = gate * conf

    out = q + gate * mem
    out = jnp.where(valid, out, q)

    mu = jnp.mean(out, axis=1, keepdims=True)
    d0 = out - mu
    var = jnp.mean(d0 * d0, axis=1, keepdims=True)
    out_ref[...] = d0 * lax.rsqrt(var + _EPS) * g_ref[...] + be_ref[...]


def _phase_f(query, retrieved, top_vals, Wq, bq, Wk, bk, Wv, bv, Wo, bo,
             Wg, bg, ln_gamma, ln_beta, interpret=False):
    row = lambda x: x.reshape(1, -1)
    return pl.pallas_call(
        _fusion_kernel,
        out_shape=jax.ShapeDtypeStruct((_B, _D), jnp.float32),
        interpret=interpret,
    )(query, retrieved, top_vals, Wq, row(bq), Wk, row(bk), Wv, row(bv),
      Wo, row(bo), row(Wg[:_D, 0]), row(Wg[_D:, 0]), row(bg),
      row(ln_gamma), row(ln_beta))


def kernel(query, memory_keys, memory_values, Wq, bq, Wk, bk, Wv, bv,
           Wo, bo, Wg, bg, ln_gamma, ln_beta):
    keys_p = jnp.pad(memory_keys, ((0, _MP - _M), (0, 0)))

    gmax8, gmax256, qn = _phase_a(query, keys_p)
    gmax = jnp.transpose(gmax256, (1, 0, 2)).reshape(_B, _NG)
    flat_ids, gid = _phase_b(gmax)

    cand = _sc_gather(gmax8.reshape(_NT * _B, _FPT),
                      flat_ids.reshape(_B * _K), rows_per_buf=64)
    fid, rows64 = _phase_d1(cand.reshape(_B, _K * _FPT), gid)

    dots, nsq = _sc_rowdot(keys_p, rows64.reshape(_B * 64),
                           qn.reshape(_B * _D))
    top_vals, top_idx = _phase_d2(dots.reshape(_B, 64),
                                  nsq.reshape(_B, 64), fid)

    retrieved = _sc_gather(memory_values, top_idx.reshape(_B * _K),
                           rows_per_buf=128)

    return _phase_f(query, retrieved.reshape(_B, _K, _D), top_vals,
                    Wq, bq, Wk, bk, Wv, bv, Wo, bo, Wg, bg,
                    ln_gamma, ln_beta)
